# baseline (speedup 1.0000x reference)
import functools

import jax
import jax.numpy as jnp
from jax import lax
from jax.experimental import pallas as pl
from jax.experimental.pallas import tpu as pltpu
from jax.experimental.pallas import tpu_sc as plsc

N = 10000
E = 320000
F = 128
H = 128
BN_EPS = 1e-5

EC = 128
NW = 32
NBUF = 3
RPW = 81
ROWS_P = NW * RPW
EPAD = ROWS_P * EC - E
NSUB = 16
NPAD = 10112
RPS = NPAD // NSUB
LAST_VALID = N - 15 * RPS


def _make_sc_kernel():
    mesh = plsc.VectorSubcoreMesh(core_axis_name="c", subcore_axis_name="s")

    @functools.partial(
        pl.kernel,
        out_type=(
            jax.ShapeDtypeStruct((N, F), jnp.float32),
            jax.ShapeDtypeStruct((N, F), jnp.float32),
        ),
        mesh=mesh,
        scratch_types=[
            *[pltpu.VMEM((EC,), jnp.int32) for _ in range(NBUF)],
            *[pltpu.VMEM((EC,), jnp.int32) for _ in range(NBUF)],
            *[pltpu.VMEM((EC, F), jnp.float32) for _ in range(NBUF)],
            pltpu.VMEM_SHARED((NPAD, F), jnp.float32),
            *[pltpu.SemaphoreType.DMA for _ in range(2 * NBUF)],
        ],
    )
    def sc_aggr(src_hbm, dst_hbm, x_hbm, out0, out1, *rest):
        sidx = rest[:NBUF]
        didx = rest[NBUF:2 * NBUF]
        rows = rest[2 * NBUF:3 * NBUF]
        aggr = rest[3 * NBUF]
        isem = rest[3 * NBUF + 1:3 * NBUF + 1 + NBUF]
        gsem = rest[3 * NBUF + 1 + NBUF:]
        c = lax.axis_index("c")
        s = lax.axis_index("s")
        w = c * NSUB + s

        def zrow(i, carry):
            for j in range(F // 16):
                rows[0][i, pl.ds(j * 16, 16)] = jnp.zeros((16,), jnp.float32)
            return carry

        lax.fori_loop(0, EC, zrow, 0)
        for k in range(RPS // EC):
            pltpu.sync_copy(rows[0], aggr.at[pl.ds(s * RPS + k * EC, EC)])
        pltpu.sync_copy(
            rows[0].at[pl.ds(0, RPS - (RPS // EC) * EC)],
            aggr.at[pl.ds(s * RPS + (RPS // EC) * EC, RPS - (RPS // EC) * EC)],
        )
        plsc.subcore_barrier()

        base = w * RPW

        def load_idx(chunk, b):
            off = (base + chunk) * EC
            pltpu.async_copy(src_hbm.at[pl.ds(off, EC)], sidx[b], isem[b])
            pltpu.async_copy(dst_hbm.at[pl.ds(off, EC)], didx[b], isem[b])

        def wait_idx(b):
            pltpu.make_async_copy(src_hbm.at[pl.ds(0, EC)], sidx[b], isem[b]).wait()
            pltpu.make_async_copy(dst_hbm.at[pl.ds(0, EC)], didx[b], isem[b]).wait()

        def gather(b):
            pltpu.async_copy(x_hbm.at[sidx[b]], rows[b], gsem[b])

        def wait_gather(b):
            pltpu.make_async_copy(x_hbm.at[sidx[b]], rows[b], gsem[b]).wait()

        for b in range(NBUF):
            load_idx(b, b)
        for b in range(NBUF - 1):
            wait_idx(b)
            gather(b)

        def body(g, carry):
            for b in range(NBUF):
                cur = NBUF * g + b
                b2 = (b + NBUF - 1) % NBUF

                @pl.when(cur + NBUF - 1 < RPW)
                def _():
                    wait_idx(b2)
                    gather(b2)

                wait_gather(b)
                pltpu.sync_copy(rows[b], aggr.at[didx[b]], add=True)

                @pl.when(cur + NBUF < RPW)
                def _():
                    load_idx(cur + NBUF, b)

            return carry

        lax.fori_loop(0, RPW // NBUF, body, 0)
        plsc.subcore_barrier()

        out = [out0, out1]
        for ci in range(2):
            @pl.when(c == ci)
            def _(ci=ci):
                @pl.when(s < NSUB - 1)
                def _():
                    pltpu.sync_copy(
                        aggr.at[pl.ds(s * RPS, RPS)],
                        out[ci].at[pl.ds(s * RPS, RPS)],
                    )

                @pl.when(s == NSUB - 1)
                def _():
                    pltpu.sync_copy(
                        aggr.at[pl.ds((NSUB - 1) * RPS, LAST_VALID)],
                        out[ci].at[pl.ds((NSUB - 1) * RPS, LAST_VALID)],
                    )

    return sc_aggr


_sc_aggr = _make_sc_kernel()

BLKR = 1000
NB = N // BLKR


def _tc_body(x_ref, p0_ref, p1_ref, w1_ref, w2_ref, prm_ref, out_ref, h1s, stat):
    p = pl.program_id(0)
    b = pl.program_id(1)

    @pl.when(p == 0)
    def _():
        a = x_ref[...] + p0_ref[...] + p1_ref[...]
        h1 = jnp.maximum(
            jnp.dot(a, w1_ref[...], preferred_element_type=jnp.float32)
            + prm_ref[0:1, :],
            0.0,
        )
        h1s[pl.ds(b * BLKR, BLKR), :] = h1
        s1 = jnp.sum(h1, axis=0, keepdims=True)
        s2 = jnp.sum(h1 * h1, axis=0, keepdims=True)

        @pl.when(b == 0)
        def _():
            stat[0:1, :] = s1
            stat[1:2, :] = s2

        @pl.when(b > 0)
        def _():
            stat[0:1, :] = stat[0:1, :] + s1
            stat[1:2, :] = stat[1:2, :] + s2

    @pl.when(p == 1)
    def _():
        @pl.when(b == 0)
        def _():
            mean = stat[0:1, :] * (1.0 / N)
            var = stat[1:2, :] * (1.0 / N) - mean * mean
            rstd = lax.rsqrt(var + BN_EPS)
            scale = prm_ref[1:2, :] * rstd
            stat[2:3, :] = scale
            stat[3:4, :] = prm_ref[2:3, :] - mean * scale

        h1 = h1s[pl.ds(b * BLKR, BLKR), :]
        h2 = h1 * stat[2:3, :] + stat[3:4, :]
        out_ref[...] = (
            jnp.dot(h2, w2_ref[...], preferred_element_type=jnp.float32)
            + prm_ref[3:4, :]
        )


def _tc_mlp(x, p0, p1, W1, W2, prm):
    return pl.pallas_call(
        _tc_body,
        grid=(2, NB),
        in_specs=[
            pl.BlockSpec((BLKR, F), lambda p, b: (jnp.where(p == 0, b, 0), 0)),
            pl.BlockSpec((BLKR, F), lambda p, b: (jnp.where(p == 0, b, 0), 0)),
            pl.BlockSpec((BLKR, F), lambda p, b: (jnp.where(p == 0, b, 0), 0)),
            pl.BlockSpec((F, H), lambda p, b: (0, 0)),
            pl.BlockSpec((H, H), lambda p, b: (0, 0)),
            pl.BlockSpec((4, H), lambda p, b: (0, 0)),
        ],
        out_specs=pl.BlockSpec((BLKR, H), lambda p, b: (jnp.where(p == 0, 0, b), 0)),
        out_shape=jax.ShapeDtypeStruct((N, H), jnp.float32),
        scratch_shapes=[
            pltpu.VMEM((N, H), jnp.float32),
            pltpu.VMEM((8, 128), jnp.float32),
        ],
    )(x, p0, p1, W1, W2, prm)


def kernel(x, edge_index, W1, b1, gamma, beta, W2, b2):
    src = jnp.concatenate([edge_index[0], jnp.zeros((EPAD,), jnp.int32)])
    dst = jnp.concatenate([edge_index[1], jnp.full((EPAD,), N, jnp.int32)])
    p0, p1 = _sc_aggr(src, dst, x)
    prm = jnp.stack([b1, gamma, beta, b2])
    return _tc_mlp(x, p0, p1, W1, W2, prm)


# speedup vs baseline: 1.3732x; 1.3732x over previous
import functools

import jax
import jax.numpy as jnp
from jax import lax
from jax.experimental import pallas as pl
from jax.experimental.pallas import tpu as pltpu
from jax.experimental.pallas import tpu_sc as plsc

N = 10000
E = 320000
F = 128
H = 128
BN_EPS = 1e-5

EC = 128
NW = 32
NBUF = 2
RPW = 80
ROWS_P = NW * RPW
EPAD = ROWS_P * EC - E
NSUB = 16
NPAD = 10112
RPS = NPAD // NSUB
LAST_VALID = N - 15 * RPS


def _make_sc_kernel():
    mesh = plsc.VectorSubcoreMesh(core_axis_name="c", subcore_axis_name="s")

    @functools.partial(
        pl.kernel,
        out_type=(
            jax.ShapeDtypeStruct((N, F), jnp.float32),
            jax.ShapeDtypeStruct((N, F), jnp.float32),
        ),
        mesh=mesh,
        scratch_types=[
            *[pltpu.VMEM((EC,), jnp.int32) for _ in range(NBUF)],
            *[pltpu.VMEM((EC,), jnp.int32) for _ in range(NBUF)],
            *[pltpu.VMEM((EC, F), jnp.float32) for _ in range(NBUF)],
            pltpu.VMEM_SHARED((NPAD, F), jnp.float32),
            *[pltpu.SemaphoreType.DMA for _ in range(NBUF)],
        ],
    )
    def sc_aggr(src_hbm, dst_hbm, x_hbm, out0, out1, *rest):
        sidx = rest[:NBUF]
        didx = rest[NBUF:2 * NBUF]
        rows = rest[2 * NBUF:3 * NBUF]
        aggr = rest[3 * NBUF]
        gsem = rest[3 * NBUF + 1:]
        c = lax.axis_index("c")
        s = lax.axis_index("s")
        w = c * NSUB + s

        def zrow(i, carry):
            for j in range(F // 16):
                rows[0][i, pl.ds(j * 16, 16)] = jnp.zeros((16,), jnp.float32)
            return carry

        lax.fori_loop(0, EC, zrow, 0)
        for k in range(RPS // EC):
            pltpu.sync_copy(rows[0], aggr.at[pl.ds(s * RPS + k * EC, EC)])
        pltpu.sync_copy(
            rows[0].at[pl.ds(0, RPS - (RPS // EC) * EC)],
            aggr.at[pl.ds(s * RPS + (RPS // EC) * EC, RPS - (RPS // EC) * EC)],
        )
        plsc.subcore_barrier()

        base = w * RPW

        def load_and_gather(chunk, b):
            off = (base + chunk) * EC
            pltpu.sync_copy(src_hbm.at[pl.ds(off, EC)], sidx[b])
            pltpu.sync_copy(dst_hbm.at[pl.ds(off, EC)], didx[b])
            pltpu.async_copy(x_hbm.at[sidx[b]], rows[b], gsem[b])

        def wait_gather(b):
            pltpu.make_async_copy(x_hbm.at[sidx[b]], rows[b], gsem[b]).wait()

        load_and_gather(0, 0)

        def body(g, carry):
            for b in range(NBUF):
                cur = NBUF * g + b

                @pl.when(cur + 1 < RPW)
                def _():
                    load_and_gather(cur + 1, (b + 1) % NBUF)

                wait_gather(b)
                pltpu.sync_copy(rows[b], aggr.at[didx[b]], add=True)

            return carry

        lax.fori_loop(0, RPW // NBUF, body, 0)
        plsc.subcore_barrier()

        out = [out0, out1]
        for ci in range(2):
            @pl.when(c == ci)
            def _(ci=ci):
                @pl.when(s < NSUB - 1)
                def _():
                    pltpu.sync_copy(
                        aggr.at[pl.ds(s * RPS, RPS)],
                        out[ci].at[pl.ds(s * RPS, RPS)],
                    )

                @pl.when(s == NSUB - 1)
                def _():
                    pltpu.sync_copy(
                        aggr.at[pl.ds((NSUB - 1) * RPS, LAST_VALID)],
                        out[ci].at[pl.ds((NSUB - 1) * RPS, LAST_VALID)],
                    )

    return sc_aggr


_sc_aggr = _make_sc_kernel()

BLKR = 1000
NB = N // BLKR


def _tc_body(x_ref, p0_ref, p1_ref, w1_ref, w2_ref, prm_ref, out_ref, h1s, stat):
    p = pl.program_id(0)
    b = pl.program_id(1)

    @pl.when(p == 0)
    def _():
        a = x_ref[...] + p0_ref[...] + p1_ref[...]
        h1 = jnp.maximum(
            jnp.dot(a, w1_ref[...], preferred_element_type=jnp.float32)
            + prm_ref[0:1, :],
            0.0,
        )
        h1s[pl.ds(b * BLKR, BLKR), :] = h1
        s1 = jnp.sum(h1, axis=0, keepdims=True)
        s2 = jnp.sum(h1 * h1, axis=0, keepdims=True)

        @pl.when(b == 0)
        def _():
            stat[0:1, :] = s1
            stat[1:2, :] = s2

        @pl.when(b > 0)
        def _():
            stat[0:1, :] = stat[0:1, :] + s1
            stat[1:2, :] = stat[1:2, :] + s2

    @pl.when(p == 1)
    def _():
        @pl.when(b == 0)
        def _():
            mean = stat[0:1, :] * (1.0 / N)
            var = stat[1:2, :] * (1.0 / N) - mean * mean
            rstd = lax.rsqrt(var + BN_EPS)
            scale = prm_ref[1:2, :] * rstd
            stat[2:3, :] = scale
            stat[3:4, :] = prm_ref[2:3, :] - mean * scale

        h1 = h1s[pl.ds(b * BLKR, BLKR), :]
        h2 = h1 * stat[2:3, :] + stat[3:4, :]
        out_ref[...] = (
            jnp.dot(h2, w2_ref[...], preferred_element_type=jnp.float32)
            + prm_ref[3:4, :]
        )


def _tc_mlp(x, p0, p1, W1, W2, prm):
    return pl.pallas_call(
        _tc_body,
        grid=(2, NB),
        in_specs=[
            pl.BlockSpec((BLKR, F), lambda p, b: (jnp.where(p == 0, b, 0), 0)),
            pl.BlockSpec((BLKR, F), lambda p, b: (jnp.where(p == 0, b, 0), 0)),
            pl.BlockSpec((BLKR, F), lambda p, b: (jnp.where(p == 0, b, 0), 0)),
            pl.BlockSpec((F, H), lambda p, b: (0, 0)),
            pl.BlockSpec((H, H), lambda p, b: (0, 0)),
            pl.BlockSpec((4, H), lambda p, b: (0, 0)),
        ],
        out_specs=pl.BlockSpec((BLKR, H), lambda p, b: (jnp.where(p == 0, 0, b), 0)),
        out_shape=jax.ShapeDtypeStruct((N, H), jnp.float32),
        scratch_shapes=[
            pltpu.VMEM((N, H), jnp.float32),
            pltpu.VMEM((8, 128), jnp.float32),
        ],
    )(x, p0, p1, W1, W2, prm)


def kernel(x, edge_index, W1, b1, gamma, beta, W2, b2):
    src = jnp.concatenate([edge_index[0], jnp.zeros((EPAD,), jnp.int32)])
    dst = jnp.concatenate([edge_index[1], jnp.full((EPAD,), N, jnp.int32)])
    p0, p1 = _sc_aggr(src, dst, x)
    prm = jnp.stack([b1, gamma, beta, b2])
    return _tc_mlp(x, p0, p1, W1, W2, prm)


# speedup vs baseline: 3.5685x; 2.5987x over previous
import functools

import jax
import jax.numpy as jnp
from jax import lax
from jax.experimental import pallas as pl
from jax.experimental.pallas import tpu as pltpu
from jax.experimental.pallas import tpu_sc as plsc

N = 10000
E = 320000
F = 128
H = 128
BN_EPS = 1e-5

EC = 128
NW = 32
NBUF = 2
RPW = 80
ROWS_P = NW * RPW
EPAD = ROWS_P * EC - E
NSUB = 16
NPAD = 10112
RPS = NPAD // NSUB
LAST_VALID = N - 15 * RPS


def _make_sc_kernel():
    mesh = plsc.VectorSubcoreMesh(core_axis_name="c", subcore_axis_name="s")

    @functools.partial(
        pl.kernel,
        out_type=(
            jax.ShapeDtypeStruct((N, F), jnp.float32),
            jax.ShapeDtypeStruct((N, F), jnp.float32),
        ),
        mesh=mesh,
        scratch_types=[
            *[pltpu.VMEM((EC,), jnp.int32) for _ in range(NBUF)],
            *[pltpu.VMEM((EC,), jnp.int32) for _ in range(NBUF)],
            *[pltpu.VMEM((EC, F), jnp.float32) for _ in range(NBUF)],
            pltpu.VMEM_SHARED((NPAD, F), jnp.float32),
            *[pltpu.SemaphoreType.DMA for _ in range(NBUF)],
        ],
    )
    def sc_aggr(src_hbm, dst_hbm, x_hbm, out0, out1, *rest):
        sidx = rest[:NBUF]
        didx = rest[NBUF:2 * NBUF]
        rows = rest[2 * NBUF:3 * NBUF]
        aggr = rest[3 * NBUF]
        gsem = rest[3 * NBUF + 1:]
        c = lax.axis_index("c")
        s = lax.axis_index("s")
        w = c * NSUB + s

        def zrow(i, carry):
            for j in range(F // 16):
                rows[0][i, pl.ds(j * 16, 16)] = jnp.zeros((16,), jnp.float32)
            return carry

        lax.fori_loop(0, EC, zrow, 0)
        for k in range(RPS // EC):
            pltpu.sync_copy(rows[0], aggr.at[pl.ds(s * RPS + k * EC, EC)])
        pltpu.sync_copy(
            rows[0].at[pl.ds(0, RPS - (RPS // EC) * EC)],
            aggr.at[pl.ds(s * RPS + (RPS // EC) * EC, RPS - (RPS // EC) * EC)],
        )
        plsc.subcore_barrier()

        base = w * RPW

        def load_and_gather(chunk, b):
            off = (base + chunk) * EC
            pltpu.sync_copy(src_hbm.at[pl.ds(off, EC)], sidx[b])
            pltpu.sync_copy(dst_hbm.at[pl.ds(off, EC)], didx[b])
            pltpu.async_copy(x_hbm.at[sidx[b]], rows[b], gsem[b])

        def wait_gather(b):
            pltpu.make_async_copy(x_hbm.at[sidx[b]], rows[b], gsem[b]).wait()

        load_and_gather(0, 0)

        def body(g, carry):
            for b in range(NBUF):
                cur = NBUF * g + b

                @pl.when(cur + 1 < RPW)
                def _():
                    load_and_gather(cur + 1, (b + 1) % NBUF)

                wait_gather(b)
                pltpu.sync_copy(rows[b], aggr.at[didx[b]], add=True)

            return carry

        lax.fori_loop(0, RPW // NBUF, body, 0)
        plsc.subcore_barrier()

        out = [out0, out1]
        for ci in range(2):
            @pl.when(c == ci)
            def _(ci=ci):
                @pl.when(s < NSUB - 1)
                def _():
                    pltpu.sync_copy(
                        aggr.at[pl.ds(s * RPS, RPS)],
                        out[ci].at[pl.ds(s * RPS, RPS)],
                    )

                @pl.when(s == NSUB - 1)
                def _():
                    pltpu.sync_copy(
                        aggr.at[pl.ds((NSUB - 1) * RPS, LAST_VALID)],
                        out[ci].at[pl.ds((NSUB - 1) * RPS, LAST_VALID)],
                    )

    return sc_aggr


_sc_aggr = _make_sc_kernel()

BLKR = 1000
NB = N // BLKR


def _tc_body(x_ref, p0_ref, p1_ref, w1_ref, w2_ref, prm_ref, out_ref, h1s, stat):
    p = pl.program_id(0)
    b = pl.program_id(1)

    @pl.when(p == 0)
    def _():
        a = x_ref[...] + p0_ref[...] + p1_ref[...]
        h1 = jnp.maximum(
            jnp.dot(a, w1_ref[...], preferred_element_type=jnp.float32)
            + prm_ref[0:1, :],
            0.0,
        )
        h1s[pl.ds(b * BLKR, BLKR), :] = h1
        s1 = jnp.sum(h1, axis=0, keepdims=True)
        s2 = jnp.sum(h1 * h1, axis=0, keepdims=True)

        @pl.when(b == 0)
        def _():
            stat[0:1, :] = s1
            stat[1:2, :] = s2

        @pl.when(b > 0)
        def _():
            stat[0:1, :] = stat[0:1, :] + s1
            stat[1:2, :] = stat[1:2, :] + s2

    @pl.when(p == 1)
    def _():
        @pl.when(b == 0)
        def _():
            mean = stat[0:1, :] * (1.0 / N)
            var = stat[1:2, :] * (1.0 / N) - mean * mean
            rstd = lax.rsqrt(var + BN_EPS)
            scale = prm_ref[1:2, :] * rstd
            stat[2:3, :] = scale
            stat[3:4, :] = prm_ref[2:3, :] - mean * scale

        h1 = h1s[pl.ds(b * BLKR, BLKR), :]
        h2 = h1 * stat[2:3, :] + stat[3:4, :]
        out_ref[...] = (
            jnp.dot(h2, w2_ref[...], preferred_element_type=jnp.float32)
            + prm_ref[3:4, :]
        )


def _tc_mlp(x, p0, p1, W1, W2, prm):
    return pl.pallas_call(
        _tc_body,
        grid=(2, NB),
        in_specs=[
            pl.BlockSpec((BLKR, F), lambda p, b: (jnp.where(p == 0, b, 0), 0)),
            pl.BlockSpec((BLKR, F), lambda p, b: (jnp.where(p == 0, b, 0), 0)),
            pl.BlockSpec((BLKR, F), lambda p, b: (jnp.where(p == 0, b, 0), 0)),
            pl.BlockSpec((F, H), lambda p, b: (0, 0)),
            pl.BlockSpec((H, H), lambda p, b: (0, 0)),
            pl.BlockSpec((4, H), lambda p, b: (0, 0)),
        ],
        out_specs=pl.BlockSpec((BLKR, H), lambda p, b: (jnp.where(p == 0, 0, b), 0)),
        out_shape=jax.ShapeDtypeStruct((N, H), jnp.float32),
        scratch_shapes=[
            pltpu.VMEM((N, H), jnp.float32),
            pltpu.VMEM((8, 128), jnp.float32),
        ],
    )(x, p0, p1, W1, W2, prm)


def kernel(x, edge_index, W1, b1, gamma, beta, W2, b2):
    pad_iota = jnp.arange(EPAD, dtype=jnp.int32)
    src = jnp.concatenate([edge_index[0], pad_iota % N])
    dst = jnp.concatenate([edge_index[1], N + pad_iota % (NPAD - N)])
    p0, p1 = _sc_aggr(src, dst, x)
    prm = jnp.stack([b1, gamma, beta, b2])
    return _tc_mlp(x, p0, p1, W1, W2, prm)


# speedup vs baseline: 4.2385x; 1.1878x over previous
import functools

import jax
import jax.numpy as jnp
from jax import lax
from jax.experimental import pallas as pl
from jax.experimental.pallas import tpu as pltpu
from jax.experimental.pallas import tpu_sc as plsc

N = 10000
E = 320000
F = 128
H = 128
BN_EPS = 1e-5

EC = 128
NW = 32
NBUF = 3
RPW = 81
ROWS_P = NW * RPW
EPAD = ROWS_P * EC - E
NSUB = 16
NPAD = 10112
RPS = NPAD // NSUB
LAST_VALID = N - 15 * RPS


def _make_sc_kernel():
    mesh = plsc.VectorSubcoreMesh(core_axis_name="c", subcore_axis_name="s")

    @functools.partial(
        pl.kernel,
        out_type=(
            jax.ShapeDtypeStruct((N, F), jnp.float32),
            jax.ShapeDtypeStruct((N, F), jnp.float32),
        ),
        mesh=mesh,
        scratch_types=[
            *[pltpu.VMEM((EC,), jnp.int32) for _ in range(NBUF)],
            *[pltpu.VMEM((EC,), jnp.int32) for _ in range(NBUF)],
            *[pltpu.VMEM((EC, F), jnp.float32) for _ in range(NBUF)],
            pltpu.VMEM_SHARED((NPAD, F), jnp.float32),
            *[pltpu.SemaphoreType.DMA for _ in range(2 * NBUF)],
        ],
    )
    def sc_aggr(src_hbm, dst_hbm, x_hbm, out0, out1, *rest):
        sidx = rest[:NBUF]
        didx = rest[NBUF:2 * NBUF]
        rows = rest[2 * NBUF:3 * NBUF]
        aggr = rest[3 * NBUF]
        isem = rest[3 * NBUF + 1:3 * NBUF + 1 + NBUF]
        gsem = rest[3 * NBUF + 1 + NBUF:]
        c = lax.axis_index("c")
        s = lax.axis_index("s")
        w = c * NSUB + s

        def zrow(i, carry):
            for j in range(F // 16):
                rows[0][i, pl.ds(j * 16, 16)] = jnp.zeros((16,), jnp.float32)
            return carry

        lax.fori_loop(0, EC, zrow, 0)
        for k in range(RPS // EC):
            pltpu.sync_copy(rows[0], aggr.at[pl.ds(s * RPS + k * EC, EC)])
        pltpu.sync_copy(
            rows[0].at[pl.ds(0, RPS - (RPS // EC) * EC)],
            aggr.at[pl.ds(s * RPS + (RPS // EC) * EC, RPS - (RPS // EC) * EC)],
        )
        plsc.subcore_barrier()

        base = w * RPW

        def load_idx(chunk, b):
            off = (base + chunk) * EC
            pltpu.async_copy(src_hbm.at[pl.ds(off, EC)], sidx[b], isem[b])
            pltpu.async_copy(dst_hbm.at[pl.ds(off, EC)], didx[b], isem[b])

        def wait_idx(b):
            pltpu.make_async_copy(src_hbm.at[pl.ds(0, EC)], sidx[b], isem[b]).wait()
            pltpu.make_async_copy(dst_hbm.at[pl.ds(0, EC)], didx[b], isem[b]).wait()

        def gather(b):
            pltpu.async_copy(x_hbm.at[sidx[b]], rows[b], gsem[b])

        def wait_gather(b):
            pltpu.make_async_copy(x_hbm.at[sidx[b]], rows[b], gsem[b]).wait()

        for b in range(NBUF):
            load_idx(b, b)
        for b in range(NBUF - 1):
            wait_idx(b)
            gather(b)

        def body(g, carry):
            for b in range(NBUF):
                cur = NBUF * g + b
                b2 = (b + NBUF - 1) % NBUF

                @pl.when(cur + NBUF - 1 < RPW)
                def _():
                    wait_idx(b2)
                    gather(b2)

                wait_gather(b)
                pltpu.sync_copy(rows[b], aggr.at[didx[b]], add=True)

                @pl.when(cur + NBUF < RPW)
                def _():
                    load_idx(cur + NBUF, b)

            return carry

        lax.fori_loop(0, RPW // NBUF, body, 0)
        plsc.subcore_barrier()

        out = [out0, out1]
        for ci in range(2):
            @pl.when(c == ci)
            def _(ci=ci):
                @pl.when(s < NSUB - 1)
                def _():
                    pltpu.sync_copy(
                        aggr.at[pl.ds(s * RPS, RPS)],
                        out[ci].at[pl.ds(s * RPS, RPS)],
                    )

                @pl.when(s == NSUB - 1)
                def _():
                    pltpu.sync_copy(
                        aggr.at[pl.ds((NSUB - 1) * RPS, LAST_VALID)],
                        out[ci].at[pl.ds((NSUB - 1) * RPS, LAST_VALID)],
                    )

    return sc_aggr


_sc_aggr = _make_sc_kernel()

BLKR = 1000
NB = N // BLKR


def _tc_body(x_ref, p0_ref, p1_ref, w1_ref, w2_ref, prm_ref, out_ref, h1s, stat):
    p = pl.program_id(0)
    b = pl.program_id(1)

    @pl.when(p == 0)
    def _():
        a = x_ref[...] + p0_ref[...] + p1_ref[...]
        h1 = jnp.maximum(
            jnp.dot(a, w1_ref[...], preferred_element_type=jnp.float32)
            + prm_ref[0:1, :],
            0.0,
        )
        h1s[pl.ds(b * BLKR, BLKR), :] = h1
        s1 = jnp.sum(h1, axis=0, keepdims=True)
        s2 = jnp.sum(h1 * h1, axis=0, keepdims=True)

        @pl.when(b == 0)
        def _():
            stat[0:1, :] = s1
            stat[1:2, :] = s2

        @pl.when(b > 0)
        def _():
            stat[0:1, :] = stat[0:1, :] + s1
            stat[1:2, :] = stat[1:2, :] + s2

    @pl.when(p == 1)
    def _():
        @pl.when(b == 0)
        def _():
            mean = stat[0:1, :] * (1.0 / N)
            var = stat[1:2, :] * (1.0 / N) - mean * mean
            rstd = lax.rsqrt(var + BN_EPS)
            scale = prm_ref[1:2, :] * rstd
            stat[2:3, :] = scale
            stat[3:4, :] = prm_ref[2:3, :] - mean * scale

        h1 = h1s[pl.ds(b * BLKR, BLKR), :]
        h2 = h1 * stat[2:3, :] + stat[3:4, :]
        out_ref[...] = (
            jnp.dot(h2, w2_ref[...], preferred_element_type=jnp.float32)
            + prm_ref[3:4, :]
        )


def _tc_mlp(x, p0, p1, W1, W2, prm):
    return pl.pallas_call(
        _tc_body,
        grid=(2, NB),
        in_specs=[
            pl.BlockSpec((BLKR, F), lambda p, b: (jnp.where(p == 0, b, 0), 0)),
            pl.BlockSpec((BLKR, F), lambda p, b: (jnp.where(p == 0, b, 0), 0)),
            pl.BlockSpec((BLKR, F), lambda p, b: (jnp.where(p == 0, b, 0), 0)),
            pl.BlockSpec((F, H), lambda p, b: (0, 0)),
            pl.BlockSpec((H, H), lambda p, b: (0, 0)),
            pl.BlockSpec((4, H), lambda p, b: (0, 0)),
        ],
        out_specs=pl.BlockSpec((BLKR, H), lambda p, b: (jnp.where(p == 0, 0, b), 0)),
        out_shape=jax.ShapeDtypeStruct((N, H), jnp.float32),
        scratch_shapes=[
            pltpu.VMEM((N, H), jnp.float32),
            pltpu.VMEM((8, 128), jnp.float32),
        ],
    )(x, p0, p1, W1, W2, prm)


def kernel(x, edge_index, W1, b1, gamma, beta, W2, b2):
    pad_iota = jnp.arange(EPAD, dtype=jnp.int32)
    src = jnp.concatenate([edge_index[0], pad_iota % N])
    dst = jnp.concatenate([edge_index[1], N + pad_iota % (NPAD - N)])
    p0, p1 = _sc_aggr(src, dst, x)
    prm = jnp.stack([b1, gamma, beta, b2])
    return _tc_mlp(x, p0, p1, W1, W2, prm)


# speedup vs baseline: 4.3008x; 1.0147x over previous
import functools

import jax
import jax.numpy as jnp
from jax import lax
from jax.experimental import pallas as pl
from jax.experimental.pallas import tpu as pltpu
from jax.experimental.pallas import tpu_sc as plsc

N = 10000
E = 320000
F = 128
H = 128
BN_EPS = 1e-5

EC = 128
NW = 32
NBUF = 3
ROWS = E // EC
BASE_ROWS = ROWS // NW
EXTRA = ROWS - BASE_ROWS * NW
NSUB = 16
NPAD = 10112
RPS = NPAD // NSUB
LAST_VALID = N - 15 * RPS


def _make_sc_kernel():
    mesh = plsc.VectorSubcoreMesh(core_axis_name="c", subcore_axis_name="s")

    @functools.partial(
        pl.kernel,
        out_type=(
            jax.ShapeDtypeStruct((N, F), jnp.float32),
            jax.ShapeDtypeStruct((N, F), jnp.float32),
        ),
        mesh=mesh,
        scratch_types=[
            *[pltpu.VMEM((EC,), jnp.int32) for _ in range(NBUF)],
            *[pltpu.VMEM((EC,), jnp.int32) for _ in range(NBUF)],
            *[pltpu.VMEM((EC, F), jnp.float32) for _ in range(NBUF)],
            pltpu.VMEM_SHARED((NPAD, F), jnp.float32),
            *[pltpu.SemaphoreType.DMA for _ in range(2 * NBUF)],
        ],
    )
    def sc_aggr(src_hbm, dst_hbm, x_hbm, out0, out1, *rest):
        sidx = rest[:NBUF]
        didx = rest[NBUF:2 * NBUF]
        rows = rest[2 * NBUF:3 * NBUF]
        aggr = rest[3 * NBUF]
        isem = rest[3 * NBUF + 1:3 * NBUF + 1 + NBUF]
        gsem = rest[3 * NBUF + 1 + NBUF:]
        c = lax.axis_index("c")
        s = lax.axis_index("s")
        w = c * NSUB + s

        def zrow(i, carry):
            for j in range(F // 16):
                rows[0][i, pl.ds(j * 16, 16)] = jnp.zeros((16,), jnp.float32)
            return carry

        lax.fori_loop(0, EC, zrow, 0)
        for k in range(RPS // EC):
            pltpu.sync_copy(rows[0], aggr.at[pl.ds(s * RPS + k * EC, EC)])
        pltpu.sync_copy(
            rows[0].at[pl.ds(0, RPS - (RPS // EC) * EC)],
            aggr.at[pl.ds(s * RPS + (RPS // EC) * EC, RPS - (RPS // EC) * EC)],
        )
        plsc.subcore_barrier()

        base = w * BASE_ROWS + jnp.minimum(w, EXTRA)
        cnt = BASE_ROWS + jnp.where(w < EXTRA, 1, 0)

        def load_idx(chunk, b):
            off = (base + chunk) * EC
            pltpu.async_copy(src_hbm.at[pl.ds(off, EC)], sidx[b], isem[b])
            pltpu.async_copy(dst_hbm.at[pl.ds(off, EC)], didx[b], isem[b])

        def wait_idx(b):
            pltpu.make_async_copy(src_hbm.at[pl.ds(0, EC)], sidx[b], isem[b]).wait()
            pltpu.make_async_copy(dst_hbm.at[pl.ds(0, EC)], didx[b], isem[b]).wait()

        def gather(b):
            pltpu.async_copy(x_hbm.at[sidx[b]], rows[b], gsem[b])

        def wait_gather(b):
            pltpu.make_async_copy(x_hbm.at[sidx[b]], rows[b], gsem[b]).wait()

        for b in range(NBUF):
            load_idx(b, b)
        for b in range(NBUF - 1):
            wait_idx(b)
            gather(b)

        def body(g, carry):
            for b in range(NBUF):
                cur = NBUF * g + b
                b2 = (b + NBUF - 1) % NBUF

                @pl.when(cur + NBUF - 1 < cnt)
                def _():
                    wait_idx(b2)
                    gather(b2)

                wait_gather(b)
                pltpu.sync_copy(rows[b], aggr.at[didx[b]], add=True)

                @pl.when(cur + NBUF < cnt)
                def _():
                    load_idx(cur + NBUF, b)

            return carry

        lax.fori_loop(0, BASE_ROWS // NBUF, body, 0)
        tb = BASE_ROWS % NBUF

        @pl.when(cnt > BASE_ROWS)
        def _():
            wait_gather(tb)
            pltpu.sync_copy(rows[tb], aggr.at[didx[tb]], add=True)

        plsc.subcore_barrier()

        out = [out0, out1]
        for ci in range(2):
            @pl.when(c == ci)
            def _(ci=ci):
                @pl.when(s < NSUB - 1)
                def _():
                    pltpu.sync_copy(
                        aggr.at[pl.ds(s * RPS, RPS)],
                        out[ci].at[pl.ds(s * RPS, RPS)],
                    )

                @pl.when(s == NSUB - 1)
                def _():
                    pltpu.sync_copy(
                        aggr.at[pl.ds((NSUB - 1) * RPS, LAST_VALID)],
                        out[ci].at[pl.ds((NSUB - 1) * RPS, LAST_VALID)],
                    )

    return sc_aggr


_sc_aggr = _make_sc_kernel()

BLKR = 1000
NB = N // BLKR


def _tc_body(x_ref, p0_ref, p1_ref, w1_ref, w2_ref, prm_ref, out_ref, h1s, stat):
    p = pl.program_id(0)
    b = pl.program_id(1)

    @pl.when(p == 0)
    def _():
        a = x_ref[...] + p0_ref[...] + p1_ref[...]
        h1 = jnp.maximum(
            jnp.dot(a, w1_ref[...], preferred_element_type=jnp.float32)
            + prm_ref[0:1, :],
            0.0,
        )
        h1s[pl.ds(b * BLKR, BLKR), :] = h1
        s1 = jnp.sum(h1, axis=0, keepdims=True)
        s2 = jnp.sum(h1 * h1, axis=0, keepdims=True)

        @pl.when(b == 0)
        def _():
            stat[0:1, :] = s1
            stat[1:2, :] = s2

        @pl.when(b > 0)
        def _():
            stat[0:1, :] = stat[0:1, :] + s1
            stat[1:2, :] = stat[1:2, :] + s2

    @pl.when(p == 1)
    def _():
        @pl.when(b == 0)
        def _():
            mean = stat[0:1, :] * (1.0 / N)
            var = stat[1:2, :] * (1.0 / N) - mean * mean
            rstd = lax.rsqrt(var + BN_EPS)
            scale = prm_ref[1:2, :] * rstd
            stat[2:3, :] = scale
            stat[3:4, :] = prm_ref[2:3, :] - mean * scale

        h1 = h1s[pl.ds(b * BLKR, BLKR), :]
        h2 = h1 * stat[2:3, :] + stat[3:4, :]
        out_ref[...] = (
            jnp.dot(h2, w2_ref[...], preferred_element_type=jnp.float32)
            + prm_ref[3:4, :]
        )


def _tc_mlp(x, p0, p1, W1, W2, prm):
    return pl.pallas_call(
        _tc_body,
        grid=(2, NB),
        in_specs=[
            pl.BlockSpec((BLKR, F), lambda p, b: (jnp.where(p == 0, b, 0), 0)),
            pl.BlockSpec((BLKR, F), lambda p, b: (jnp.where(p == 0, b, 0), 0)),
            pl.BlockSpec((BLKR, F), lambda p, b: (jnp.where(p == 0, b, 0), 0)),
            pl.BlockSpec((F, H), lambda p, b: (0, 0)),
            pl.BlockSpec((H, H), lambda p, b: (0, 0)),
            pl.BlockSpec((4, H), lambda p, b: (0, 0)),
        ],
        out_specs=pl.BlockSpec((BLKR, H), lambda p, b: (jnp.where(p == 0, 0, b), 0)),
        out_shape=jax.ShapeDtypeStruct((N, H), jnp.float32),
        scratch_shapes=[
            pltpu.VMEM((N, H), jnp.float32),
            pltpu.VMEM((8, 128), jnp.float32),
        ],
    )(x, p0, p1, W1, W2, prm)


def kernel(x, edge_index, W1, b1, gamma, beta, W2, b2):
    src = edge_index[0]
    dst = edge_index[1]
    p0, p1 = _sc_aggr(src, dst, x)
    prm = jnp.stack([b1, gamma, beta, b2])
    return _tc_mlp(x, p0, p1, W1, W2, prm)


# speedup vs baseline: 4.6336x; 1.0774x over previous
import functools

import jax
import jax.numpy as jnp
from jax import lax
from jax.experimental import pallas as pl
from jax.experimental.pallas import tpu as pltpu
from jax.experimental.pallas import tpu_sc as plsc

N = 10000
E = 320000
F = 128
H = 128
BN_EPS = 1e-5

EC = 128
NW = 32
NBUF = 3
ROWS = E // EC
BASE_ROWS = ROWS // NW
EXTRA = ROWS - BASE_ROWS * NW
NSUB = 16
NPAD = 10112
RPS = NPAD // NSUB
LAST_VALID = N - 15 * RPS


def _make_sc_kernel():
    mesh = plsc.VectorSubcoreMesh(core_axis_name="c", subcore_axis_name="s")

    @functools.partial(
        pl.kernel,
        out_type=(
            jax.ShapeDtypeStruct((N, F), jnp.float32),
            jax.ShapeDtypeStruct((N, F), jnp.float32),
        ),
        mesh=mesh,
        scratch_types=[
            *[pltpu.VMEM((EC,), jnp.int32) for _ in range(NBUF)],
            *[pltpu.VMEM((EC,), jnp.int32) for _ in range(NBUF)],
            *[pltpu.VMEM((EC, F), jnp.float32) for _ in range(NBUF)],
            pltpu.VMEM_SHARED((NPAD, F), jnp.float32),
            *[pltpu.SemaphoreType.DMA for _ in range(2 * NBUF)],
        ],
    )
    def sc_aggr(src_hbm, dst_hbm, x_hbm, out0, out1, *rest):
        sidx = rest[:NBUF]
        didx = rest[NBUF:2 * NBUF]
        rows = rest[2 * NBUF:3 * NBUF]
        aggr = rest[3 * NBUF]
        isem = rest[3 * NBUF + 1:3 * NBUF + 1 + NBUF]
        gsem = rest[3 * NBUF + 1 + NBUF:]
        c = lax.axis_index("c")
        s = lax.axis_index("s")
        w = c * NSUB + s

        def zrow(i, carry):
            for j in range(F // 16):
                rows[0][i, pl.ds(j * 16, 16)] = jnp.zeros((16,), jnp.float32)
            return carry

        lax.fori_loop(0, EC, zrow, 0)
        for k in range(RPS // EC):
            pltpu.sync_copy(rows[0], aggr.at[pl.ds(s * RPS + k * EC, EC)])
        pltpu.sync_copy(
            rows[0].at[pl.ds(0, RPS - (RPS // EC) * EC)],
            aggr.at[pl.ds(s * RPS + (RPS // EC) * EC, RPS - (RPS // EC) * EC)],
        )
        plsc.subcore_barrier()

        base = w * BASE_ROWS + jnp.minimum(w, EXTRA)
        cnt = BASE_ROWS + jnp.where(w < EXTRA, 1, 0)

        def load_idx(chunk, b):
            off = (base + chunk) * EC
            pltpu.async_copy(src_hbm.at[pl.ds(off, EC)], sidx[b], isem[b])
            pltpu.async_copy(dst_hbm.at[pl.ds(off, EC)], didx[b], isem[b])

        def wait_idx(b):
            pltpu.make_async_copy(src_hbm.at[pl.ds(0, EC)], sidx[b], isem[b]).wait()
            pltpu.make_async_copy(dst_hbm.at[pl.ds(0, EC)], didx[b], isem[b]).wait()

        def gather(b):
            pltpu.async_copy(x_hbm.at[sidx[b]], rows[b], gsem[b])

        def wait_gather(b):
            pltpu.make_async_copy(x_hbm.at[sidx[b]], rows[b], gsem[b]).wait()

        for b in range(NBUF):
            load_idx(b, b)
        for b in range(NBUF - 1):
            wait_idx(b)
            gather(b)

        def body(g, carry):
            for b in range(NBUF):
                cur = NBUF * g + b
                b2 = (b + NBUF - 1) % NBUF

                @pl.when(cur + NBUF - 1 < cnt)
                def _():
                    wait_idx(b2)
                    gather(b2)

                wait_gather(b)
                pltpu.sync_copy(rows[b], aggr.at[didx[b]], add=True)

                @pl.when(cur + NBUF < cnt)
                def _():
                    load_idx(cur + NBUF, b)

            return carry

        lax.fori_loop(0, BASE_ROWS // NBUF, body, 0)
        tb = BASE_ROWS % NBUF

        @pl.when(cnt > BASE_ROWS)
        def _():
            wait_gather(tb)
            pltpu.sync_copy(rows[tb], aggr.at[didx[tb]], add=True)

        plsc.subcore_barrier()

        out = [out0, out1]
        for ci in range(2):
            @pl.when(c == ci)
            def _(ci=ci):
                @pl.when(s < NSUB - 1)
                def _():
                    pltpu.sync_copy(
                        aggr.at[pl.ds(s * RPS, RPS)],
                        out[ci].at[pl.ds(s * RPS, RPS)],
                    )

                @pl.when(s == NSUB - 1)
                def _():
                    pltpu.sync_copy(
                        aggr.at[pl.ds((NSUB - 1) * RPS, LAST_VALID)],
                        out[ci].at[pl.ds((NSUB - 1) * RPS, LAST_VALID)],
                    )

    return sc_aggr


_sc_aggr = _make_sc_kernel()

BLKR = 1000
NB = N // BLKR


BLKE = 32000


def _split_body(ei_ref, s_ref, d_ref):
    s_ref[...] = ei_ref[0, :]
    d_ref[...] = ei_ref[1, :]


def _split(edge_index):
    return pl.pallas_call(
        _split_body,
        out_shape=[
            jax.ShapeDtypeStruct((E,), jnp.int32),
            jax.ShapeDtypeStruct((E,), jnp.int32),
        ],
    )(edge_index)


def _tc_body(x_ref, p0_ref, p1_ref, w1_ref, w2_ref, prm_ref, out_ref, h1s, stat):
    p = pl.program_id(0)
    b = pl.program_id(1)

    @pl.when(p == 0)
    def _():
        a = x_ref[...] + p0_ref[...] + p1_ref[...]
        h1 = jnp.maximum(
            jnp.dot(a, w1_ref[...], preferred_element_type=jnp.float32)
            + prm_ref[0:1, :],
            0.0,
        )
        h1s[pl.ds(b * BLKR, BLKR), :] = h1
        s1 = jnp.sum(h1, axis=0, keepdims=True)
        s2 = jnp.sum(h1 * h1, axis=0, keepdims=True)

        @pl.when(b == 0)
        def _():
            stat[0:1, :] = s1
            stat[1:2, :] = s2

        @pl.when(b > 0)
        def _():
            stat[0:1, :] = stat[0:1, :] + s1
            stat[1:2, :] = stat[1:2, :] + s2

    @pl.when(p == 1)
    def _():
        @pl.when(b == 0)
        def _():
            mean = stat[0:1, :] * (1.0 / N)
            var = stat[1:2, :] * (1.0 / N) - mean * mean
            rstd = lax.rsqrt(var + BN_EPS)
            scale = prm_ref[1:2, :] * rstd
            stat[2:3, :] = scale
            stat[3:4, :] = prm_ref[2:3, :] - mean * scale

        h1 = h1s[pl.ds(b * BLKR, BLKR), :]
        h2 = h1 * stat[2:3, :] + stat[3:4, :]
        out_ref[...] = (
            jnp.dot(h2, w2_ref[...], preferred_element_type=jnp.float32)
            + prm_ref[3:4, :]
        )


def _tc_mlp(x, p0, p1, W1, W2, prm):
    return pl.pallas_call(
        _tc_body,
        grid=(2, NB),
        in_specs=[
            pl.BlockSpec((BLKR, F), lambda p, b: (jnp.where(p == 0, b, 0), 0)),
            pl.BlockSpec((BLKR, F), lambda p, b: (jnp.where(p == 0, b, 0), 0)),
            pl.BlockSpec((BLKR, F), lambda p, b: (jnp.where(p == 0, b, 0), 0)),
            pl.BlockSpec((F, H), lambda p, b: (0, 0)),
            pl.BlockSpec((H, H), lambda p, b: (0, 0)),
            pl.BlockSpec((4, H), lambda p, b: (0, 0)),
        ],
        out_specs=pl.BlockSpec((BLKR, H), lambda p, b: (jnp.where(p == 0, 0, b), 0)),
        out_shape=jax.ShapeDtypeStruct((N, H), jnp.float32),
        scratch_shapes=[
            pltpu.VMEM((N, H), jnp.float32),
            pltpu.VMEM((8, 128), jnp.float32),
        ],
    )(x, p0, p1, W1, W2, prm)


def kernel(x, edge_index, W1, b1, gamma, beta, W2, b2):
    src, dst = _split(edge_index)
    p0, p1 = _sc_aggr(src, dst, x)
    prm = jnp.stack([b1, gamma, beta, b2])
    return _tc_mlp(x, p0, p1, W1, W2, prm)


# speedup vs baseline: 4.7186x; 1.0183x over previous
import functools

import jax
import jax.numpy as jnp
from jax import lax
from jax.experimental import pallas as pl
from jax.experimental.pallas import tpu as pltpu
from jax.experimental.pallas import tpu_sc as plsc

N = 10000
E = 320000
F = 128
H = 128
BN_EPS = 1e-5

EC = 128
NW = 32
NBUF = 3
ROWS = E // EC
BASE_ROWS = ROWS // NW
EXTRA = ROWS - BASE_ROWS * NW
NSUB = 16
NPAD = 10112
RPS = NPAD // NSUB
LAST_VALID = N - 15 * RPS


def _make_sc_kernel():
    mesh = plsc.VectorSubcoreMesh(core_axis_name="c", subcore_axis_name="s")

    @functools.partial(
        pl.kernel,
        out_type=(
            jax.ShapeDtypeStruct((N, F), jnp.float32),
            jax.ShapeDtypeStruct((N, F), jnp.float32),
        ),
        mesh=mesh,
        scratch_types=[
            *[pltpu.VMEM((EC,), jnp.int32) for _ in range(NBUF)],
            *[pltpu.VMEM((EC,), jnp.int32) for _ in range(NBUF)],
            *[pltpu.VMEM((EC, F), jnp.float32) for _ in range(NBUF)],
            pltpu.VMEM_SHARED((NPAD, F), jnp.float32),
            *[pltpu.SemaphoreType.DMA for _ in range(2 * NBUF)],
        ],
    )
    def sc_aggr(src_hbm, dst_hbm, x_hbm, out0, out1, *rest):
        sidx = rest[:NBUF]
        didx = rest[NBUF:2 * NBUF]
        rows = rest[2 * NBUF:3 * NBUF]
        aggr = rest[3 * NBUF]
        isem = rest[3 * NBUF + 1:3 * NBUF + 1 + NBUF]
        gsem = rest[3 * NBUF + 1 + NBUF:]
        c = lax.axis_index("c")
        s = lax.axis_index("s")
        w = c * NSUB + s

        def zrow(i, carry):
            for j in range(F // 16):
                rows[0][i, pl.ds(j * 16, 16)] = jnp.zeros((16,), jnp.float32)
            return carry

        lax.fori_loop(0, EC, zrow, 0)
        TAIL = RPS - (RPS // EC) * EC

        @pl.when(c == 1)
        def _():
            for k in range(RPS // EC):
                pltpu.sync_copy(rows[0], aggr.at[pl.ds(s * RPS + k * EC, EC)])
            pltpu.sync_copy(
                rows[0].at[pl.ds(0, TAIL)],
                aggr.at[pl.ds(s * RPS + (RPS // EC) * EC, TAIL)],
            )

        @pl.when((c == 0) & (s < NSUB - 1))
        def _():
            for k in range(RPS // EC):
                pltpu.sync_copy(
                    x_hbm.at[pl.ds(s * RPS + k * EC, EC)],
                    aggr.at[pl.ds(s * RPS + k * EC, EC)],
                )
            pltpu.sync_copy(
                x_hbm.at[pl.ds(s * RPS + (RPS // EC) * EC, TAIL)],
                aggr.at[pl.ds(s * RPS + (RPS // EC) * EC, TAIL)],
            )

        @pl.when((c == 0) & (s == NSUB - 1))
        def _():
            base15 = (NSUB - 1) * RPS
            for k in range(RPS // EC):
                pltpu.sync_copy(
                    x_hbm.at[pl.ds(base15 + k * EC, EC)],
                    aggr.at[pl.ds(base15 + k * EC, EC)],
                )
            pltpu.sync_copy(
                x_hbm.at[pl.ds(base15 + (RPS // EC) * EC, LAST_VALID - (RPS // EC) * EC)],
                aggr.at[pl.ds(base15 + (RPS // EC) * EC, LAST_VALID - (RPS // EC) * EC)],
            )
            pltpu.sync_copy(
                rows[0].at[pl.ds(0, NPAD - N)],
                aggr.at[pl.ds(N, NPAD - N)],
            )

        plsc.subcore_barrier()

        base = w * BASE_ROWS + jnp.minimum(w, EXTRA)
        cnt = BASE_ROWS + jnp.where(w < EXTRA, 1, 0)

        def load_idx(chunk, b):
            off = (base + chunk) * EC
            pltpu.async_copy(src_hbm.at[pl.ds(off, EC)], sidx[b], isem[b])
            pltpu.async_copy(dst_hbm.at[pl.ds(off, EC)], didx[b], isem[b])

        def wait_idx(b):
            pltpu.make_async_copy(src_hbm.at[pl.ds(0, EC)], sidx[b], isem[b]).wait()
            pltpu.make_async_copy(dst_hbm.at[pl.ds(0, EC)], didx[b], isem[b]).wait()

        def gather(b):
            pltpu.async_copy(x_hbm.at[sidx[b]], rows[b], gsem[b])

        def wait_gather(b):
            pltpu.make_async_copy(x_hbm.at[sidx[b]], rows[b], gsem[b]).wait()

        for b in range(NBUF):
            load_idx(b, b)
        for b in range(NBUF - 1):
            wait_idx(b)
            gather(b)

        def body(g, carry):
            for b in range(NBUF):
                cur = NBUF * g + b
                b2 = (b + NBUF - 1) % NBUF

                @pl.when(cur + NBUF - 1 < cnt)
                def _():
                    wait_idx(b2)
                    gather(b2)

                wait_gather(b)
                pltpu.sync_copy(rows[b], aggr.at[didx[b]], add=True)

                @pl.when(cur + NBUF < cnt)
                def _():
                    load_idx(cur + NBUF, b)

            return carry

        lax.fori_loop(0, BASE_ROWS // NBUF, body, 0)
        tb = BASE_ROWS % NBUF

        @pl.when(cnt > BASE_ROWS)
        def _():
            wait_gather(tb)
            pltpu.sync_copy(rows[tb], aggr.at[didx[tb]], add=True)

        plsc.subcore_barrier()

        out = [out0, out1]
        for ci in range(2):
            @pl.when(c == ci)
            def _(ci=ci):
                @pl.when(s < NSUB - 1)
                def _():
                    pltpu.sync_copy(
                        aggr.at[pl.ds(s * RPS, RPS)],
                        out[ci].at[pl.ds(s * RPS, RPS)],
                    )

                @pl.when(s == NSUB - 1)
                def _():
                    pltpu.sync_copy(
                        aggr.at[pl.ds((NSUB - 1) * RPS, LAST_VALID)],
                        out[ci].at[pl.ds((NSUB - 1) * RPS, LAST_VALID)],
                    )

    return sc_aggr


_sc_aggr = _make_sc_kernel()

BLKR = 2000
NB = N // BLKR


BLKE = 32000


def _split_body(ei_ref, s_ref, d_ref):
    s_ref[...] = ei_ref[0, :]
    d_ref[...] = ei_ref[1, :]


def _split(edge_index):
    return pl.pallas_call(
        _split_body,
        out_shape=[
            jax.ShapeDtypeStruct((E,), jnp.int32),
            jax.ShapeDtypeStruct((E,), jnp.int32),
        ],
    )(edge_index)


def _tc_body(p0_ref, p1_ref, w1_ref, w2_ref, prm_ref, out_ref, h1s, stat):
    p = pl.program_id(0)
    b = pl.program_id(1)

    @pl.when(p == 0)
    def _():
        a = p0_ref[...] + p1_ref[...]
        h1 = jnp.maximum(
            jnp.dot(a, w1_ref[...], preferred_element_type=jnp.float32)
            + prm_ref[0:1, :],
            0.0,
        )
        h1s[pl.ds(b * BLKR, BLKR), :] = h1
        s1 = jnp.sum(h1, axis=0, keepdims=True)
        s2 = jnp.sum(h1 * h1, axis=0, keepdims=True)

        @pl.when(b == 0)
        def _():
            stat[0:1, :] = s1
            stat[1:2, :] = s2

        @pl.when(b > 0)
        def _():
            stat[0:1, :] = stat[0:1, :] + s1
            stat[1:2, :] = stat[1:2, :] + s2

    @pl.when(p == 1)
    def _():
        @pl.when(b == 0)
        def _():
            mean = stat[0:1, :] * (1.0 / N)
            var = stat[1:2, :] * (1.0 / N) - mean * mean
            rstd = lax.rsqrt(var + BN_EPS)
            scale = prm_ref[1:2, :] * rstd
            stat[2:3, :] = scale
            stat[3:4, :] = prm_ref[2:3, :] - mean * scale

        h1 = h1s[pl.ds(b * BLKR, BLKR), :]
        h2 = h1 * stat[2:3, :] + stat[3:4, :]
        out_ref[...] = (
            jnp.dot(h2, w2_ref[...], preferred_element_type=jnp.float32)
            + prm_ref[3:4, :]
        )


def _tc_mlp(p0, p1, W1, W2, prm):
    return pl.pallas_call(
        _tc_body,
        grid=(2, NB),
        in_specs=[
            pl.BlockSpec((BLKR, F), lambda p, b: (jnp.where(p == 0, b, 0), 0)),
            pl.BlockSpec((BLKR, F), lambda p, b: (jnp.where(p == 0, b, 0), 0)),
            pl.BlockSpec((F, H), lambda p, b: (0, 0)),
            pl.BlockSpec((H, H), lambda p, b: (0, 0)),
            pl.BlockSpec((4, H), lambda p, b: (0, 0)),
        ],
        out_specs=pl.BlockSpec((BLKR, H), lambda p, b: (jnp.where(p == 0, 0, b), 0)),
        out_shape=jax.ShapeDtypeStruct((N, H), jnp.float32),
        scratch_shapes=[
            pltpu.VMEM((N, H), jnp.float32),
            pltpu.VMEM((8, 128), jnp.float32),
        ],
    )(p0, p1, W1, W2, prm)


def kernel(x, edge_index, W1, b1, gamma, beta, W2, b2):
    src, dst = _split(edge_index)
    p0, p1 = _sc_aggr(src, dst, x)
    prm = jnp.stack([b1, gamma, beta, b2])
    return _tc_mlp(p0, p1, W1, W2, prm)
